# baseline (speedup 1.0000x reference)
import functools

import jax
import jax.numpy as jnp
from jax import lax
from jax.experimental import pallas as pl
from jax.experimental.pallas import tpu as pltpu
from jax.experimental.pallas import tpu_sc as plsc

B = 4096
D = 8192
L = 16
NC = 2
NS = 16
NW = NC * NS
BPW = B // NW
K = 4
NCHUNK = BPW // K

_mesh = plsc.VectorSubcoreMesh(core_axis_name="c", subcore_axis_name="s")


@functools.partial(
    pl.kernel,
    mesh=_mesh,
    out_type=jax.ShapeDtypeStruct((B,), jnp.float32),
    compiler_params=pltpu.CompilerParams(needs_layout_passes=False),
    scratch_types=[
        pltpu.VMEM((NCHUNK, K), jnp.int32),
        pltpu.VMEM((NCHUNK, K), jnp.int32),
        pltpu.VMEM((K, D), jnp.float32),
        pltpu.VMEM((K, D), jnp.float32),
        pltpu.VMEM((BPW,), jnp.float32),
        pltpu.VMEM((L, L), jnp.float32),
        pltpu.SemaphoreType.DMA,
        pltpu.SemaphoreType.DMA,
    ],
)
def _slim_body(user_hbm, item_hbm, a_hbm, w_hbm, out_hbm,
               uidx, iidx, a_rows, w_rows, res, acc_buf, sem_a, sem_w):
    wid = lax.axis_index("s") * NC + lax.axis_index("c")
    pltpu.sync_copy(user_hbm.at[pl.ds(wid * NCHUNK, NCHUNK)], uidx)
    pltpu.sync_copy(item_hbm.at[pl.ds(wid * NCHUNK, NCHUNK)], iidx)

    lane_iota = lax.iota(jnp.int32, L)
    cpg = L // K

    def group_body(g, carry):
        for c in range(cpg):
            gc = g * cpg + c
            ca = pltpu.async_copy(a_hbm.at[uidx.at[gc]], a_rows, sem_a)
            cw = pltpu.async_copy(w_hbm.at[iidx.at[gc]], w_rows, sem_w)
            ca.wait()
            cw.wait()
            for r in range(K):
                def inner(j, acc):
                    av = a_rows[r, pl.ds(j * L, L)]
                    wv = w_rows[r, pl.ds(j * L, L)]
                    return acc + av * wv
                acc = lax.fori_loop(0, D // L, inner,
                                    jnp.zeros((L,), jnp.float32))
                acc_buf[c * K + r] = acc
        totals = jnp.zeros((L,), jnp.float32)
        for c in range(L):
            col_idx = jnp.full((L,), c, jnp.int32)
            totals = totals + plsc.load_gather(acc_buf, [lane_iota, col_idx])
        res[pl.ds(g * L, L)] = totals
        return carry

    lax.fori_loop(0, BPW // L, group_body, 0)
    pltpu.sync_copy(res, out_hbm.at[pl.ds(wid * BPW, BPW)])


def kernel(user, item, A, W):
    user2 = user.astype(jnp.int32).reshape(NW * NCHUNK, K)
    item2 = item.astype(jnp.int32).reshape(NW * NCHUNK, K)
    return _slim_body(user2, item2, A, W)


# speedup vs baseline: 2.6525x; 2.6525x over previous
import functools

import jax
import jax.numpy as jnp
from jax import lax
from jax.experimental import pallas as pl
from jax.experimental.pallas import tpu as pltpu
from jax.experimental.pallas import tpu_sc as plsc

B = 4096
D = 8192
L = 16
NC = 2
NS = 16
NW = NC * NS
BPW = B // NW
K = 2
NCHUNK = BPW // K
CPG = L // K
UNROLL = 4

_mesh = plsc.VectorSubcoreMesh(core_axis_name="c", subcore_axis_name="s")


def _row_dot(a_ref, w_ref, r):
    def inner(j, accs):
        base = j * (4 * UNROLL * L)
        new = []
        for q in range(4):
            acc = accs[q]
            for u in range(UNROLL):
                off = base + (q * UNROLL + u) * L
                acc = acc + a_ref[r, pl.ds(off, L)] * w_ref[r, pl.ds(off, L)]
            new.append(acc)
        return tuple(new)

    zeros = jnp.zeros((L,), jnp.float32)
    accs = lax.fori_loop(0, D // (4 * UNROLL * L), inner,
                         (zeros, zeros, zeros, zeros))
    return (accs[0] + accs[1]) + (accs[2] + accs[3])


@functools.partial(
    pl.kernel,
    mesh=_mesh,
    out_type=jax.ShapeDtypeStruct((B,), jnp.float32),
    compiler_params=pltpu.CompilerParams(needs_layout_passes=False),
    scratch_types=[
        pltpu.VMEM((NCHUNK, K), jnp.int32),
        pltpu.VMEM((NCHUNK, K), jnp.int32),
        pltpu.VMEM((K, D), jnp.float32),
        pltpu.VMEM((K, D), jnp.float32),
        pltpu.VMEM((K, D), jnp.float32),
        pltpu.VMEM((K, D), jnp.float32),
        pltpu.VMEM((BPW,), jnp.float32),
        pltpu.VMEM((L, L), jnp.float32),
        pltpu.SemaphoreType.DMA,
        pltpu.SemaphoreType.DMA,
        pltpu.SemaphoreType.DMA,
        pltpu.SemaphoreType.DMA,
    ],
)
def _slim_body(user_hbm, item_hbm, a_hbm, w_hbm, out_hbm,
               uidx, iidx, a0, a1, w0, w1, res, acc_buf,
               sem_a0, sem_a1, sem_w0, sem_w1):
    wid = lax.axis_index("s") * NC + lax.axis_index("c")
    pltpu.sync_copy(user_hbm.at[pl.ds(wid * NCHUNK, NCHUNK)], uidx)
    pltpu.sync_copy(item_hbm.at[pl.ds(wid * NCHUNK, NCHUNK)], iidx)

    a_bufs = (a0, a1)
    w_bufs = (w0, w1)
    sems_a = (sem_a0, sem_a1)
    sems_w = (sem_w0, sem_w1)
    lane_iota = lax.iota(jnp.int32, L)

    def start(gc, slot):
        pltpu.async_copy(a_hbm.at[uidx.at[gc]], a_bufs[slot], sems_a[slot])
        pltpu.async_copy(w_hbm.at[iidx.at[gc]], w_bufs[slot], sems_w[slot])

    def wait(gc, slot):
        pltpu.make_async_copy(
            a_hbm.at[uidx.at[gc]], a_bufs[slot], sems_a[slot]).wait()
        pltpu.make_async_copy(
            w_hbm.at[iidx.at[gc]], w_bufs[slot], sems_w[slot]).wait()

    start(0, 0)
    start(1, 1)

    def group_body(g, carry):
        for c in range(CPG):
            slot = c % 2
            gc = g * CPG + c
            wait(gc, slot)
            for r in range(K):
                acc = _row_dot(a_bufs[slot], w_bufs[slot], r)
                acc_buf[c * K + r] = acc

            @pl.when(gc + 2 < NCHUNK)
            def _():
                start(gc + 2, slot)

        totals = jnp.zeros((L,), jnp.float32)
        for c in range(L):
            col_idx = jnp.full((L,), c, jnp.int32)
            totals = totals + plsc.load_gather(acc_buf, [lane_iota, col_idx])
        res[pl.ds(g * L, L)] = totals
        return carry

    lax.fori_loop(0, BPW // L, group_body, 0)
    pltpu.sync_copy(res, out_hbm.at[pl.ds(wid * BPW, BPW)])


def kernel(user, item, A, W):
    user2 = user.astype(jnp.int32).reshape(NW * NCHUNK, K)
    item2 = item.astype(jnp.int32).reshape(NW * NCHUNK, K)
    return _slim_body(user2, item2, A, W)


# speedup vs baseline: 3.1113x; 1.1730x over previous
import functools

import jax
import jax.numpy as jnp
from jax import lax
from jax.experimental import pallas as pl
from jax.experimental.pallas import tpu as pltpu
from jax.experimental.pallas import tpu_sc as plsc

B = 4096
D = 8192
L = 16
NC = 2
NS = 16
NW = NC * NS
BPW = B // NW
NSLOT = 4
UNROLL = 4

_mesh = plsc.VectorSubcoreMesh(core_axis_name="c", subcore_axis_name="s")


def _row_dot(a_ref, w_ref):
    def inner(j, accs):
        base = j * (4 * UNROLL * L)
        new = []
        for q in range(4):
            acc = accs[q]
            for u in range(UNROLL):
                off = base + (q * UNROLL + u) * L
                acc = acc + a_ref[0, pl.ds(off, L)] * w_ref[0, pl.ds(off, L)]
            new.append(acc)
        return tuple(new)

    zeros = jnp.zeros((L,), jnp.float32)
    accs = lax.fori_loop(0, D // (4 * UNROLL * L), inner,
                         (zeros, zeros, zeros, zeros))
    return (accs[0] + accs[1]) + (accs[2] + accs[3])


@functools.partial(
    pl.kernel,
    mesh=_mesh,
    out_type=jax.ShapeDtypeStruct((B,), jnp.float32),
    compiler_params=pltpu.CompilerParams(needs_layout_passes=False),
    scratch_types=[
        pltpu.VMEM((BPW, 1), jnp.int32),
        pltpu.VMEM((BPW, 1), jnp.int32),
        [pltpu.VMEM((1, D), jnp.float32) for _ in range(NSLOT)],
        [pltpu.VMEM((1, D), jnp.float32) for _ in range(NSLOT)],
        pltpu.VMEM((BPW,), jnp.float32),
        pltpu.VMEM((L, L), jnp.float32),
        [pltpu.SemaphoreType.DMA for _ in range(NSLOT)],
        [pltpu.SemaphoreType.DMA for _ in range(NSLOT)],
    ],
)
def _slim_body(user_hbm, item_hbm, a_hbm, w_hbm, out_hbm,
               uidx, iidx, a_bufs, w_bufs, res, acc_buf, sems_a, sems_w):
    wid = lax.axis_index("s") * NC + lax.axis_index("c")
    pltpu.sync_copy(user_hbm.at[pl.ds(wid * BPW, BPW)], uidx)
    pltpu.sync_copy(item_hbm.at[pl.ds(wid * BPW, BPW)], iidx)

    lane_iota = lax.iota(jnp.int32, L)

    def start(e, slot):
        pltpu.async_copy(a_hbm.at[uidx.at[e]], a_bufs[slot], sems_a[slot])
        pltpu.async_copy(w_hbm.at[iidx.at[e]], w_bufs[slot], sems_w[slot])

    def wait(e, slot):
        pltpu.make_async_copy(
            a_hbm.at[uidx.at[e]], a_bufs[slot], sems_a[slot]).wait()
        pltpu.make_async_copy(
            w_hbm.at[iidx.at[e]], w_bufs[slot], sems_w[slot]).wait()

    for s in range(NSLOT):
        start(s, s)

    def group_body(g, carry):
        for c in range(L):
            slot = c % NSLOT
            e = g * L + c
            wait(e, slot)
            acc_buf[c] = _row_dot(a_bufs[slot], w_bufs[slot])

            @pl.when(e + NSLOT < BPW)
            def _():
                start(e + NSLOT, slot)

        totals = jnp.zeros((L,), jnp.float32)
        for c in range(L):
            col_idx = jnp.full((L,), c, jnp.int32)
            totals = totals + plsc.load_gather(acc_buf, [lane_iota, col_idx])
        res[pl.ds(g * L, L)] = totals
        return carry

    lax.fori_loop(0, BPW // L, group_body, 0)
    pltpu.sync_copy(res, out_hbm.at[pl.ds(wid * BPW, BPW)])


def kernel(user, item, A, W):
    user2 = user.astype(jnp.int32).reshape(B, 1)
    item2 = item.astype(jnp.int32).reshape(B, 1)
    return _slim_body(user2, item2, A, W)


# speedup vs baseline: 3.1301x; 1.0060x over previous
import functools

import jax
import jax.numpy as jnp
from jax import lax
from jax.experimental import pallas as pl
from jax.experimental.pallas import tpu as pltpu
from jax.experimental.pallas import tpu_sc as plsc

B = 4096
D = 8192
L = 16
NC = 2
NS = 16
NW = NC * NS
BPW = B // NW
NSLOT = 4
UNROLL = 8

_mesh = plsc.VectorSubcoreMesh(core_axis_name="c", subcore_axis_name="s")


def _row_dot(a_ref, w_ref):
    def inner(j, accs):
        base = j * (4 * UNROLL * L)
        new = []
        for q in range(4):
            acc = accs[q]
            for u in range(UNROLL):
                off = base + (q * UNROLL + u) * L
                acc = acc + a_ref[0, pl.ds(off, L)] * w_ref[0, pl.ds(off, L)]
            new.append(acc)
        return tuple(new)

    zeros = jnp.zeros((L,), jnp.float32)
    accs = lax.fori_loop(0, D // (4 * UNROLL * L), inner,
                         (zeros, zeros, zeros, zeros))
    return (accs[0] + accs[1]) + (accs[2] + accs[3])


@functools.partial(
    pl.kernel,
    mesh=_mesh,
    out_type=jax.ShapeDtypeStruct((B,), jnp.float32),
    compiler_params=pltpu.CompilerParams(needs_layout_passes=False),
    scratch_types=[
        pltpu.VMEM((BPW, 1), jnp.int32),
        pltpu.VMEM((BPW, 1), jnp.int32),
        [pltpu.VMEM((1, D), jnp.float32) for _ in range(NSLOT)],
        [pltpu.VMEM((1, D), jnp.float32) for _ in range(NSLOT)],
        pltpu.VMEM((BPW,), jnp.float32),
        pltpu.VMEM((L, L), jnp.float32),
        [pltpu.SemaphoreType.DMA for _ in range(NSLOT)],
        [pltpu.SemaphoreType.DMA for _ in range(NSLOT)],
    ],
)
def _slim_body(user_hbm, item_hbm, a_hbm, w_hbm, out_hbm,
               uidx, iidx, a_bufs, w_bufs, res, acc_buf, sems_a, sems_w):
    wid = lax.axis_index("s") * NC + lax.axis_index("c")
    pltpu.sync_copy(user_hbm.at[pl.ds(wid * BPW, BPW)], uidx)
    pltpu.sync_copy(item_hbm.at[pl.ds(wid * BPW, BPW)], iidx)

    lane_iota = lax.iota(jnp.int32, L)

    def start(e, slot):
        pltpu.async_copy(a_hbm.at[uidx.at[e]], a_bufs[slot], sems_a[slot])
        pltpu.async_copy(w_hbm.at[iidx.at[e]], w_bufs[slot], sems_w[slot])

    def wait(e, slot):
        pltpu.make_async_copy(
            a_hbm.at[uidx.at[e]], a_bufs[slot], sems_a[slot]).wait()
        pltpu.make_async_copy(
            w_hbm.at[iidx.at[e]], w_bufs[slot], sems_w[slot]).wait()

    for s in range(NSLOT):
        start(s, s)

    def group_body(g, carry):
        for c in range(L):
            slot = c % NSLOT
            e = g * L + c
            wait(e, slot)
            acc_buf[c] = _row_dot(a_bufs[slot], w_bufs[slot])

            @pl.when(e + NSLOT < BPW)
            def _():
                start(e + NSLOT, slot)

        totals = jnp.zeros((L,), jnp.float32)
        for c in range(L):
            col_idx = jnp.full((L,), c, jnp.int32)
            totals = totals + plsc.load_gather(acc_buf, [lane_iota, col_idx])
        res[pl.ds(g * L, L)] = totals
        return carry

    lax.fori_loop(0, BPW // L, group_body, 0)
    pltpu.sync_copy(res, out_hbm.at[pl.ds(wid * BPW, BPW)])


def kernel(user, item, A, W):
    user2 = user.astype(jnp.int32).reshape(B, 1)
    item2 = item.astype(jnp.int32).reshape(B, 1)
    return _slim_body(user2, item2, A, W)
